# baseline (speedup 1.0000x reference)
import functools

import jax
import jax.numpy as jnp
from jax import lax
from jax.experimental import pallas as pl
from jax.experimental.pallas import tpu as pltpu
from jax.experimental.pallas import tpu_sc as plsc

BATCH = 4096
HIST = 200
DIM = 64
NW = 32
ROWS_W = BATCH // NW
PER_W = ROWS_W * HIST
CHUNK = 128
NCHUNK = PER_W // CHUNK
NBUF = 5
NGROUP = NCHUNK // NBUF

_mesh = plsc.VectorSubcoreMesh(core_axis_name="c", subcore_axis_name="s")


def _body(table_hbm, idx_hbm, out_hbm, idx_v,
          b0, b1, b2, b3, b4,
          g0, g1, g2, g3, g4, s0, s1, s2, s3, s4):
    bufs = (b0, b1, b2, b3, b4)
    gsem = (g0, g1, g2, g3, g4)
    ssem = (s0, s1, s2, s3, s4)
    wid = lax.axis_index("s") * 2 + lax.axis_index("c")

    pltpu.sync_copy(idx_hbm.at[pl.ds(wid * PER_W, PER_W)], idx_v)

    def gather_start(j, b):
        pltpu.async_copy(
            table_hbm.at[idx_v.at[pl.ds(j * CHUNK, CHUNK)]], bufs[b], gsem[b])

    def gather_wait(j, b):
        pltpu.make_async_copy(
            table_hbm.at[idx_v.at[pl.ds(j * CHUNK, CHUNK)]], bufs[b],
            gsem[b]).wait()

    def out_dst(j):
        return out_hbm.at[pl.ds(wid * PER_W + j * CHUNK, CHUNK)]

    def store_start(j, b):
        pltpu.async_copy(bufs[b], out_dst(j), ssem[b])

    def store_wait(j, b):
        pltpu.make_async_copy(bufs[b], out_dst(j), ssem[b]).wait()

    for b in range(NBUF):
        gather_start(b, b)

    def group(g, carry):
        for b in range(NBUF):
            j = g * NBUF + b
            gather_wait(j, b)
            store_start(j, b)
            store_wait(j, b)
            gather_start(j + NBUF, b)
        return carry

    lax.fori_loop(0, NGROUP - 1, group, 0)

    for b in range(NBUF):
        j = (NGROUP - 1) * NBUF + b
        gather_wait(j, b)
        store_start(j, b)
    for b in range(NBUF):
        j = (NGROUP - 1) * NBUF + b
        store_wait(j, b)


_call = functools.partial(
    pl.kernel,
    mesh=_mesh,
    out_type=jax.ShapeDtypeStruct((BATCH * HIST, 2 * DIM), jnp.float32),
    scratch_types=(
        [pltpu.VMEM((PER_W,), jnp.int32)]
        + [pltpu.VMEM((CHUNK, 2 * DIM), jnp.float32)] * NBUF
        + [pltpu.SemaphoreType.DMA] * (2 * NBUF)
    ),
)(_body)


VOCAB = 1000000
NWIN = VOCAB // 128
TAIL0 = NWIN * 128
WPW = NWIN // NW + 1


def _bodyA(tblT_hbm, tail_hbm, out_hbm, s0, s1, d0, d1, g0, g1, p0, p1):
    sbufs = (s0, s1)
    dbufs = (d0, d1)
    gsem = (g0, g1)
    ssem = (p0, p1)
    wid = lax.axis_index("s") * 2 + lax.axis_index("c")

    @pl.when(wid == NW - 1)
    def _():
        pltpu.sync_copy(tail_hbm, out_hbm.at[pl.ds(TAIL0, VOCAB - TAIL0)])

    lane = lax.iota(jnp.int32, 16)
    skew = [(lane + j) & 15 for j in range(16)]
    dvecs = [lane + db * 16 for db in range(4)]

    def wof(k):
        return wid + NW * k

    def gather_start(k, slot):
        @pl.when(wof(k) < NWIN)
        def _():
            pltpu.async_copy(tblT_hbm.at[:, pl.ds(wof(k) * 128, 128)],
                             sbufs[slot], gsem[slot])

    def gather_wait(k, slot):
        @pl.when(wof(k) < NWIN)
        def _():
            pltpu.make_async_copy(tblT_hbm.at[:, pl.ds(wof(k) * 128, 128)],
                                  sbufs[slot], gsem[slot]).wait()

    def store_start(k, slot):
        @pl.when(wof(k) < NWIN)
        def _():
            pltpu.async_copy(dbufs[slot],
                             out_hbm.at[pl.ds(wof(k) * 128, 128)], ssem[slot])

    def store_wait(k, slot):
        @pl.when((k >= 0) & (wof(k) < NWIN))
        def _():
            pltpu.make_async_copy(dbufs[slot],
                                  out_hbm.at[pl.ds(wof(k) * 128, 128)],
                                  ssem[slot]).wait()

    gather_start(0, 0)
    gather_start(1, 1)

    def iteration(k, slot):
        gather_wait(k, slot)
        store_wait(k - 2, slot)

        @pl.when(wof(k) < NWIN)
        def _():
            @plsc.parallel_loop(0, 8, 1, unroll=2)
            def _(tb):
                t0 = tb * 16
                tvecs = [t0 + skew[j] for j in range(16)]
                for db in range(4):
                    vs = [plsc.load_gather(sbufs[slot], [dvecs[db], tvecs[j]])
                          for j in range(16)]
                    for j in range(16):
                        plsc.store_scatter(
                            dbufs[slot], [tvecs[j], dvecs[db]], vs[j])
        store_start(k, slot)
        gather_start(k + 2, slot)

    def group(g2, carry):
        iteration(2 * g2, 0)
        iteration(2 * g2 + 1, 1)
        return carry

    lax.fori_loop(0, (WPW + 1) // 2, group, 0)
    store_wait(2 * ((WPW + 1) // 2) - 2, 0)
    store_wait(2 * ((WPW + 1) // 2) - 1, 1)


_callA = functools.partial(
    pl.kernel,
    mesh=_mesh,
    compiler_params=pltpu.CompilerParams(
        needs_layout_passes=False, disable_bounds_checks=True),
    out_type=jax.ShapeDtypeStruct((VOCAB, 2 * DIM), jnp.float32),
    scratch_types=(
        [pltpu.VMEM((DIM, 128), jnp.float32)] * 2
        + [pltpu.VMEM((128, 2 * DIM), jnp.float32)] * 2
        + [pltpu.SemaphoreType.DMA] * 4
    ),
)(_bodyA)


def kernel(token_ids, embedding):
    tail128 = jnp.pad(embedding[TAIL0:], ((0, 0), (0, DIM)))
    table128 = _callA(embedding.T, tail128)
    out5 = _call(table128, token_ids.reshape(-1).astype(jnp.int32))
    return out5.reshape(BATCH, HIST, 2 * DIM)[:, :, :DIM]


# speedup vs baseline: 1.5826x; 1.5826x over previous
import functools

import jax
import jax.numpy as jnp
from jax import lax
from jax.experimental import pallas as pl
from jax.experimental.pallas import tpu as pltpu
from jax.experimental.pallas import tpu_sc as plsc

BATCH = 4096
HIST = 200
DIM = 64
NW = 32
ROWS_W = BATCH // NW
PER_W = ROWS_W * HIST
CHUNK = 128
NCHUNK = PER_W // CHUNK
NBUF = 5
NGROUP = NCHUNK // NBUF

_mesh = plsc.VectorSubcoreMesh(core_axis_name="c", subcore_axis_name="s")


def _body(table_hbm, idx_hbm, out_hbm, idx_v,
          b0, b1, b2, b3, b4,
          g0, g1, g2, g3, g4, s0, s1, s2, s3, s4):
    bufs = (b0, b1, b2, b3, b4)
    gsem = (g0, g1, g2, g3, g4)
    ssem = (s0, s1, s2, s3, s4)
    wid = lax.axis_index("s") * 2 + lax.axis_index("c")

    pltpu.sync_copy(idx_hbm.at[pl.ds(wid * PER_W, PER_W)], idx_v)

    def gather_start(j, b):
        pltpu.async_copy(
            table_hbm.at[idx_v.at[pl.ds(j * CHUNK, CHUNK)]], bufs[b], gsem[b])

    def gather_wait(j, b):
        pltpu.make_async_copy(
            table_hbm.at[idx_v.at[pl.ds(j * CHUNK, CHUNK)]], bufs[b],
            gsem[b]).wait()

    def out_dst(j):
        return out_hbm.at[pl.ds(wid * PER_W + j * CHUNK, CHUNK)]

    def store_start(j, b):
        pltpu.async_copy(bufs[b], out_dst(j), ssem[b])

    def store_wait(j, b):
        pltpu.make_async_copy(bufs[b], out_dst(j), ssem[b]).wait()

    for b in range(NBUF):
        gather_start(b, b)

    def group(g, carry):
        for b in range(NBUF):
            j = g * NBUF + b
            gather_wait(j, b)
            store_start(j, b)
            store_wait(j, b)
            gather_start(j + NBUF, b)
        return carry

    lax.fori_loop(0, NGROUP - 1, group, 0)

    for b in range(NBUF):
        j = (NGROUP - 1) * NBUF + b
        gather_wait(j, b)
        store_start(j, b)
    for b in range(NBUF):
        j = (NGROUP - 1) * NBUF + b
        store_wait(j, b)


_call = functools.partial(
    pl.kernel,
    mesh=_mesh,
    out_type=jax.ShapeDtypeStruct((BATCH * HIST, 2 * DIM), jnp.float32),
    scratch_types=(
        [pltpu.VMEM((PER_W,), jnp.int32)]
        + [pltpu.VMEM((CHUNK, 2 * DIM), jnp.float32)] * NBUF
        + [pltpu.SemaphoreType.DMA] * (2 * NBUF)
    ),
)(_body)


VOCAB = 1000000
NWIN = VOCAB // 128
TAIL0 = NWIN * 128
WPW = NWIN // NW + 1


def _bodyA(tblT_hbm, tail_hbm, out_hbm, s0, s1, d0, d1, g0, g1, p0, p1):
    sbufs = (s0, s1)
    dbufs = (d0, d1)
    gsem = (g0, g1)
    ssem = (p0, p1)
    wid = lax.axis_index("s") * 2 + lax.axis_index("c")

    @pl.when(wid == NW - 1)
    def _():
        pltpu.sync_copy(tail_hbm, out_hbm.at[pl.ds(TAIL0, VOCAB - TAIL0)])

    lane = lax.iota(jnp.int32, 16)
    skew = [(lane + j) & 15 for j in range(16)]
    dvecs = [lane + db * 16 for db in range(4)]

    def wof(k):
        return wid + NW * k

    def gather_start(k, slot):
        @pl.when(wof(k) < NWIN)
        def _():
            pltpu.async_copy(tblT_hbm.at[:, pl.ds(wof(k) * 128, 128)],
                             sbufs[slot], gsem[slot])

    def gather_wait(k, slot):
        @pl.when(wof(k) < NWIN)
        def _():
            pltpu.make_async_copy(tblT_hbm.at[:, pl.ds(wof(k) * 128, 128)],
                                  sbufs[slot], gsem[slot]).wait()

    def store_start(k, slot):
        @pl.when(wof(k) < NWIN)
        def _():
            pltpu.async_copy(dbufs[slot],
                             out_hbm.at[pl.ds(wof(k) * 128, 128)], ssem[slot])

    def store_wait(k, slot):
        @pl.when((k >= 0) & (wof(k) < NWIN))
        def _():
            pltpu.make_async_copy(dbufs[slot],
                                  out_hbm.at[pl.ds(wof(k) * 128, 128)],
                                  ssem[slot]).wait()

    gather_start(0, 0)
    gather_start(1, 1)

    def iteration(k, slot):
        gather_wait(k, slot)
        store_wait(k - 2, slot)

        @pl.when(wof(k) < NWIN)
        def _():
            @plsc.parallel_loop(0, 8, 1, unroll=1)
            def _(tb):
                t0 = tb * 16
                tvecs = [t0 + skew[j] for j in range(16)]
                for db in range(4):
                    vs = [plsc.load_gather(sbufs[slot], [dvecs[db], tvecs[j]])
                          for j in range(16)]
                    for j in range(16):
                        plsc.store_scatter(
                            dbufs[slot], [tvecs[j], dvecs[db]], vs[j])
        store_start(k, slot)
        gather_start(k + 2, slot)

    def group(g2, carry):
        iteration(2 * g2, 0)
        iteration(2 * g2 + 1, 1)
        return carry

    lax.fori_loop(0, (WPW + 1) // 2, group, 0)
    store_wait(2 * ((WPW + 1) // 2) - 2, 0)
    store_wait(2 * ((WPW + 1) // 2) - 1, 1)


_callA = functools.partial(
    pl.kernel,
    mesh=_mesh,
    compiler_params=pltpu.CompilerParams(
        needs_layout_passes=False, disable_bounds_checks=True),
    out_type=jax.ShapeDtypeStruct((VOCAB, 2 * DIM), jnp.float32),
    scratch_types=(
        [pltpu.VMEM((DIM, 128), jnp.float32)] * 2
        + [pltpu.VMEM((128, 2 * DIM), jnp.float32)] * 2
        + [pltpu.SemaphoreType.DMA] * 4
    ),
)(_bodyA)


def kernel(token_ids, embedding):
    tail128 = jnp.pad(embedding[TAIL0:], ((0, 0), (0, DIM)))
    table128 = _callA(embedding.T, tail128)
    out5 = _call(table128, token_ids.reshape(-1).astype(jnp.int32))
    return out5.reshape(BATCH, HIST, 2 * DIM)[:, :, :DIM]
